# baseline (speedup 1.0000x reference)
import jax
import jax.numpy as jnp
from jax.experimental import pallas as pl
from jax.experimental.pallas import tpu as pltpu

_B, _T, _D = 16384, 100, 64
_LANES = 128
_MROWS = (_B * _T) // _LANES
_NCHUNK = 8
_CB = _B // _NCHUNK


def _body(x_hbm, y_hbm, mask_ref, sems):
    for i in range(_NCHUNK):
        pltpu.make_async_copy(
            x_hbm.at[pl.ds(i * _CB, _CB)],
            y_hbm.at[pl.ds(i * _CB, _CB)],
            sems.at[i],
        ).start()
    mask_ref[...] = jnp.zeros_like(mask_ref)
    for i in range(_NCHUNK):
        pltpu.make_async_copy(
            x_hbm.at[pl.ds(i * _CB, _CB)],
            y_hbm.at[pl.ds(i * _CB, _CB)],
            sems.at[i],
        ).wait()


def kernel(inputs):
    y, mask2d = pl.pallas_call(
        _body,
        out_shape=(
            jax.ShapeDtypeStruct((_B, _T, _D), inputs.dtype),
            jax.ShapeDtypeStruct((_MROWS, _LANES), inputs.dtype),
        ),
        in_specs=[pl.BlockSpec(memory_space=pltpu.MemorySpace.HBM)],
        out_specs=(
            pl.BlockSpec(memory_space=pltpu.MemorySpace.HBM),
            pl.BlockSpec((_MROWS, _LANES), lambda: (0, 0)),
        ),
        scratch_shapes=[pltpu.SemaphoreType.DMA((_NCHUNK,))],
    )(inputs)
    return (y, mask2d.reshape(_B, _T, 1))


# speedup vs baseline: 15.2860x; 15.2860x over previous
import jax
import jax.numpy as jnp
from jax.experimental import pallas as pl

_B, _T, _D = 16384, 100, 64
_LANES = 128
_MROWS = (_B * _T) // _LANES
_GRID = 64
_BB = _B // _GRID
_MB = _MROWS // _GRID


def _body(x_ref, y_ref, mask_ref):
    y_ref[...] = x_ref[...]
    mask_ref[...] = jnp.zeros_like(mask_ref)


def kernel(inputs):
    y, mask2d = pl.pallas_call(
        _body,
        out_shape=(
            jax.ShapeDtypeStruct((_B, _T, _D), inputs.dtype),
            jax.ShapeDtypeStruct((_MROWS, _LANES), inputs.dtype),
        ),
        grid=(_GRID,),
        in_specs=[pl.BlockSpec((_BB, _T, _D), lambda i: (i, 0, 0))],
        out_specs=(
            pl.BlockSpec((_BB, _T, _D), lambda i: (i, 0, 0)),
            pl.BlockSpec((_MB, _LANES), lambda i: (i, 0)),
        ),
    )(inputs)
    return (y, mask2d.reshape(_B, _T, 1))
